# Initial kernel scaffold; baseline (speedup 1.0000x reference)
import jax
import jax.numpy as jnp
from jax.experimental import pallas as pl


def kernel(x, edge_index, batch, s, h, g, W_in, b_in, Wc0, as0, ad0, bc0, Wc1, as1, ad1, bc1, W1, b1, W2, b2, W3, b3):
    raise NotImplementedError("write your pallas kernel here")


# speedup vs baseline: 29.5548x; 29.5548x over previous
import jax
import jax.numpy as jnp
from jax import lax
from jax.experimental import pallas as pl
from jax.experimental.pallas import tpu as pltpu
from jax.experimental.pallas import tpu_sc as plsc

N = 50000
E = 800000
B = 8
D_IN = 128
HID = 64
HEADS = 4
HD = 16
RT = 40
RD = 16
NC = 2
NS = 16
G = 80
NGB = 79
EPAD = NS * NGB * 8 * G
NPAD = 50048
NPADX = NPAD // 8
RPT = NPAD // NS
RPTX = NPADX // NS
ZR = 136
NB = 2000
NBLK = N // NB


def _k1_body(x_ref, b_ref, s_ref, h_ref, Wx_ref, Wsh_ref, bin_ref,
             Wsrc_ref, Wdst_ref, tab_ref, dtab_ref):
    sh = jnp.concatenate([s_ref[...], h_ref[...]], axis=1)
    shproj = sh @ Wsh_ref[...]
    onehot = (b_ref[...] == lax.broadcasted_iota(jnp.int32, (1, B), 1)
              ).astype(jnp.float32)
    z = x_ref[...] @ Wx_ref[...] + onehot @ shproj + bin_ref[...]
    z = jnp.maximum(z, 0.0)
    tab_ref[0] = z @ Wsrc_ref[0]
    tab_ref[1] = z @ Wsrc_ref[1]
    dtab_ref[...] = z @ Wdst_ref[...]


def _z_from_acc(f0, f1, x0, x1, bc):
    def head(f, x, j):
        num = f[:, HD * j:HD * (j + 1)]
        den = x[:, j:j + 1] + 1e-16
        return num / den
    z = jnp.concatenate([head(f0, x0, 0), head(f0, x0, 1),
                         head(f1, x1, 0), head(f1, x1, 1)], axis=1) + bc
    return jnp.maximum(z, 0.0)


def _k2_body(f0_ref, f1_ref, x0_ref, x1_ref, bc_ref, Wsrc_ref, Wdst_ref,
             tab_ref, dtab_ref):
    z = _z_from_acc(f0_ref[0], f1_ref[0], x0_ref[0], x1_ref[0], bc_ref[...])
    tab_ref[0] = z @ Wsrc_ref[0]
    tab_ref[1] = z @ Wsrc_ref[1]
    dtab_ref[...] = z @ Wdst_ref[...]


def _k3_body(f0_ref, f1_ref, x0_ref, x1_ref, b_ref, s_ref, h_ref, g_ref,
             bc_ref, W1_ref, b1_ref, W2_ref, b2_ref, W3_ref, b3_ref,
             o_ref, pooled):
    i = pl.program_id(0)

    @pl.when(i == 0)
    def _init():
        pooled[...] = jnp.zeros_like(pooled)

    z = _z_from_acc(f0_ref[0], f1_ref[0], x0_ref[0], x1_ref[0], bc_ref[...])
    onehot = (b_ref[...] == lax.broadcasted_iota(jnp.int32, (1, B), 1)
              ).astype(jnp.float32)
    pooled[...] += lax.dot_general(onehot, z, (((0,), (0,)), ((), ())))

    @pl.when(i == NBLK - 1)
    def _final():
        comb = jnp.concatenate(
            [pooled[...], s_ref[...], h_ref[...], g_ref[...]], axis=1)
        o1 = jnp.maximum(comb @ W1_ref[...] + b1_ref[...], 0.0)
        o2 = jnp.maximum(o1 @ W2_ref[...] + b2_ref[...], 0.0)
        o_ref[...] = o2 @ W3_ref[...] + b3_ref[...]


def _edge_body(srcI_ref, dstI_ref, tab_ref, dtab_ref, outf_ref, outx_ref,
               sidx, sidx2, didx, didx8, srows, drows, orows, xrows,
               zbuf, zbufx, accf, accx):
    c = lax.axis_index("c")
    sid = lax.axis_index("s")
    lane = lax.iota(jnp.int32, 16)
    par = jnp.bitwise_and(lane, 1)
    z16 = jnp.zeros((16,), jnp.float32)
    coff = c * NPAD

    def zb(r, cy):
        zbuf[r, pl.ds(0, 16)] = z16
        zbuf[r, pl.ds(16, 16)] = z16
        return cy
    lax.fori_loop(0, ZR, zb, 0)

    def zc(r, cy):
        pltpu.sync_copy(zbuf, accf.at[pl.ds(sid * RPT + r * ZR, ZR)])
        return cy
    lax.fori_loop(0, RPT // ZR, zc, 0)

    def zx(r, cy):
        zbufx[r, pl.ds(0, 16)] = z16
        return cy
    lax.fori_loop(0, RPTX, zx, 0)
    pltpu.sync_copy(zbufx, accx.at[pl.ds(sid * RPTX, RPTX)])
    plsc.subcore_barrier()

    def chunk(gb, carry):
        pltpu.sync_copy(srcI_ref.at[sid, gb], sidx)
        pltpu.sync_copy(dstI_ref.at[sid, gb], didx)

        def addrow(j, cy):
            def addk(k, cy2):
                sl = pl.ds(k * 16, 16)
                sidx2[j, sl] = sidx[j, sl] + coff
                didx8[j, sl] = lax.shift_right_logical(didx[j, sl], 3)
                return cy2
            return lax.fori_loop(0, G // 16, addk, cy)
        lax.fori_loop(0, 8, addrow, 0)

        def group(j, cy):
            pltpu.sync_copy(tab_ref.at[sidx2.at[j]], srows)
            pltpu.sync_copy(dtab_ref.at[didx.at[j]], drows)

            def block16(bq, cy2):
                dvec = didx[j, pl.ds(bq * 16, 16)]
                base = bq * 16
                for li in range(16):
                    e = base + li
                    kk2 = 2 * jnp.bitwise_and(dvec[li], 7)
                    v = srows[e, pl.ds(24, 16)]
                    w = drows[e, pl.ds(0, 16)]
                    a0 = v[8]
                    a1 = v[9]
                    d0 = jnp.where(c == 0, w[0], w[2])
                    d1 = jnp.where(c == 0, w[1], w[3])
                    sv = jnp.where(par == 0, a0 + d0, a1 + d1)
                    sv = jnp.maximum(sv, sv * 0.2)
                    exv = jnp.exp(sv)
                    e0 = exv[0]
                    e1 = exv[1]
                    orows[e, pl.ds(0, 16)] = srows[e, pl.ds(0, 16)] * e0
                    orows[e, pl.ds(16, 16)] = srows[e, pl.ds(16, 16)] * e1
                    xrows[e, pl.ds(0, 16)] = jnp.where(
                        lane == kk2, e0, jnp.where(lane == kk2 + 1, e1, 0.0))
                return cy2
            lax.fori_loop(0, G // 16, block16, 0)

            pltpu.sync_copy(orows, accf.at[didx.at[j]], add=True)
            pltpu.sync_copy(xrows, accx.at[didx8.at[j]], add=True)
            return cy
        lax.fori_loop(0, 8, group, 0)
        return carry
    lax.fori_loop(0, NGB, chunk, 0)

    plsc.subcore_barrier()
    pltpu.sync_copy(accf.at[pl.ds(sid * RPT, RPT)],
                    outf_ref.at[c, pl.ds(sid * RPT, RPT)])
    pltpu.sync_copy(accx.at[pl.ds(sid * RPTX, RPTX)],
                    outx_ref.at[c, pl.ds(sid * RPTX, RPTX)])


def _edge_pass(srcI, dstI, tab_flat, dtab):
    mesh = plsc.VectorSubcoreMesh(core_axis_name="c", subcore_axis_name="s",
                                  num_cores=NC, num_subcores=NS)
    fn = pl.kernel(
        _edge_body,
        out_type=[
            jax.ShapeDtypeStruct((NC, NPAD, 32), jnp.float32),
            jax.ShapeDtypeStruct((NC, NPADX, 16), jnp.float32),
        ],
        mesh=mesh,
        compiler_params=pltpu.CompilerParams(use_tc_tiling_on_sc=False),
        scratch_types=[
            pltpu.VMEM((8, G), jnp.int32),
            pltpu.VMEM((8, G), jnp.int32),
            pltpu.VMEM((8, G), jnp.int32),
            pltpu.VMEM((8, G), jnp.int32),
            pltpu.VMEM((G, RT), jnp.float32),
            pltpu.VMEM((G, RD), jnp.float32),
            pltpu.VMEM((G, 32), jnp.float32),
            pltpu.VMEM((G, 16), jnp.float32),
            pltpu.VMEM((ZR, 32), jnp.float32),
            pltpu.VMEM((RPTX, 16), jnp.float32),
            pltpu.VMEM_SHARED((NPAD, 32), jnp.float32),
            pltpu.VMEM_SHARED((NPADX, 16), jnp.float32),
        ],
    )
    return fn(srcI, dstI, tab_flat, dtab)


def _node_tables_l0(x, batch2d, s, h, Wx, Wsh, b_in, Wsrc, Wdst):
    return pl.pallas_call(
        _k1_body,
        grid=(NBLK,),
        in_specs=[
            pl.BlockSpec((NB, D_IN), lambda i: (i, 0)),
            pl.BlockSpec((NB, 1), lambda i: (i, 0)),
            pl.BlockSpec((B, 32), lambda i: (0, 0)),
            pl.BlockSpec((B, 16), lambda i: (0, 0)),
            pl.BlockSpec((D_IN, HID), lambda i: (0, 0)),
            pl.BlockSpec((48, HID), lambda i: (0, 0)),
            pl.BlockSpec((1, HID), lambda i: (0, 0)),
            pl.BlockSpec((NC, HID, RT), lambda i: (0, 0, 0)),
            pl.BlockSpec((HID, RD), lambda i: (0, 0)),
        ],
        out_specs=[
            pl.BlockSpec((NC, NB, RT), lambda i: (0, i, 0)),
            pl.BlockSpec((NB, RD), lambda i: (i, 0)),
        ],
        out_shape=[
            jax.ShapeDtypeStruct((NC, NPAD, RT), jnp.float32),
            jax.ShapeDtypeStruct((NPAD, RD), jnp.float32),
        ],
    )(x, batch2d, s, h, Wx, Wsh, b_in, Wsrc, Wdst)


def _acc_specs():
    return [
        pl.BlockSpec((1, NB, 32), lambda i: (0, i, 0)),
        pl.BlockSpec((1, NB, 32), lambda i: (1, i, 0)),
        pl.BlockSpec((1, NB, 2), lambda i: (0, i, 0)),
        pl.BlockSpec((1, NB, 2), lambda i: (1, i, 0)),
    ]


def _node_tables_l1(accf, accx2, bc, Wsrc, Wdst):
    return pl.pallas_call(
        _k2_body,
        grid=(NBLK,),
        in_specs=_acc_specs() + [
            pl.BlockSpec((1, HID), lambda i: (0, 0)),
            pl.BlockSpec((NC, HID, RT), lambda i: (0, 0, 0)),
            pl.BlockSpec((HID, RD), lambda i: (0, 0)),
        ],
        out_specs=[
            pl.BlockSpec((NC, NB, RT), lambda i: (0, i, 0)),
            pl.BlockSpec((NB, RD), lambda i: (i, 0)),
        ],
        out_shape=[
            jax.ShapeDtypeStruct((NC, NPAD, RT), jnp.float32),
            jax.ShapeDtypeStruct((NPAD, RD), jnp.float32),
        ],
    )(accf, accf, accx2, accx2, bc, Wsrc, Wdst)


def _readout(accf, accx2, batch2d, s, h, g, bc, W1, b1, W2, b2, W3, b3):
    return pl.pallas_call(
        _k3_body,
        grid=(NBLK,),
        in_specs=_acc_specs() + [
            pl.BlockSpec((NB, 1), lambda i: (i, 0)),
            pl.BlockSpec((B, 32), lambda i: (0, 0)),
            pl.BlockSpec((B, 16), lambda i: (0, 0)),
            pl.BlockSpec((B, 8), lambda i: (0, 0)),
            pl.BlockSpec((1, HID), lambda i: (0, 0)),
            pl.BlockSpec((120, 256), lambda i: (0, 0)),
            pl.BlockSpec((1, 256), lambda i: (0, 0)),
            pl.BlockSpec((256, 128), lambda i: (0, 0)),
            pl.BlockSpec((1, 128), lambda i: (0, 0)),
            pl.BlockSpec((128, 1), lambda i: (0, 0)),
            pl.BlockSpec((1, 1), lambda i: (0, 0)),
        ],
        out_specs=pl.BlockSpec((B, 1), lambda i: (0, 0)),
        out_shape=jax.ShapeDtypeStruct((B, 1), jnp.float32),
        scratch_shapes=[pltpu.VMEM((B, HID), jnp.float32)],
    )(accf, accf, accx2, accx2, batch2d, s, h, g, bc,
      W1, b1, W2, b2, W3, b3)


def _layer_weights(Wc, a_s, a_d):
    Wc3 = Wc.reshape(HID, HEADS, HD)
    Asrc = jnp.einsum('khd,hd->kh', Wc3, a_s)
    Adst = jnp.einsum('khd,hd->kh', Wc3, a_d)
    z6 = jnp.zeros((HID, 6), jnp.float32)
    Wsrc = jnp.stack([
        jnp.concatenate([Wc[:, 0:32], Asrc[:, 0:2], z6], axis=1),
        jnp.concatenate([Wc[:, 32:64], Asrc[:, 2:4], z6], axis=1)], axis=0)
    Wdst = jnp.concatenate([Adst, jnp.zeros((HID, 12), jnp.float32)], axis=1)
    return Wsrc, Wdst


def kernel(x, edge_index, batch, s, h, g, W_in, b_in, Wc0, as0, ad0, bc0,
           Wc1, as1, ad1, bc1, W1, b1, W2, b2, W3, b3):
    pad = jnp.full((EPAD - E,), N, jnp.int32)
    srcI = jnp.concatenate([edge_index[0], pad]).reshape(NS, NGB, 8, G)
    dstI = jnp.concatenate([edge_index[1], pad]).reshape(NS, NGB, 8, G)
    batch2d = batch.reshape(N, 1)
    Wx = W_in[:D_IN]
    Wsh = W_in[D_IN:]
    Wsrc0, Wdst0 = _layer_weights(Wc0, as0, ad0)
    Wsrc1, Wdst1 = _layer_weights(Wc1, as1, ad1)

    tab0, dtab0 = _node_tables_l0(x, batch2d, s, h, Wx, Wsh,
                                  b_in.reshape(1, HID), Wsrc0, Wdst0)
    f0, x0 = _edge_pass(srcI, dstI, tab0.reshape(NC * NPAD, RT), dtab0)
    tab1, dtab1 = _node_tables_l1(f0, x0.reshape(NC, NPAD, 2),
                                  bc0.reshape(1, HID), Wsrc1, Wdst1)
    f1, x1 = _edge_pass(srcI, dstI, tab1.reshape(NC * NPAD, RT), dtab1)
    o = _readout(f1, x1.reshape(NC, NPAD, 2), batch2d, s, h, g,
                 bc1.reshape(1, HID), W1, b1.reshape(1, 256),
                 W2, b2.reshape(1, 128), W3, b3.reshape(1, 1))
    return o[:, 0]


# speedup vs baseline: 42.7731x; 1.4472x over previous
import jax
import jax.numpy as jnp
from jax import lax
from jax.experimental import pallas as pl
from jax.experimental.pallas import tpu as pltpu
from jax.experimental.pallas import tpu_sc as plsc

N = 50000
E = 800000
B = 8
D_IN = 128
HID = 64
HEADS = 4
HD = 16
RT = 40
RD = 16
NC = 2
NS = 16
G = 80
NGB = 79
EPAD = NS * NGB * 8 * G
NPAD = 50048
NPADX = NPAD // 8
RPT = NPAD // NS
RPTX = NPADX // NS
ZR = 68
ZRX = 23
NB = 2000
NBLK = N // NB


def _k1_body(x_ref, b_ref, s_ref, h_ref, Wx_ref, Wsh_ref, bin_ref,
             Wsrc_ref, Wdst_ref, tab_ref, dtab_ref):
    sh = jnp.concatenate([s_ref[...], h_ref[...]], axis=1)
    shproj = sh @ Wsh_ref[...]
    onehot = (b_ref[...] == lax.broadcasted_iota(jnp.int32, (1, B), 1)
              ).astype(jnp.float32)
    z = x_ref[...] @ Wx_ref[...] + onehot @ shproj + bin_ref[...]
    z = jnp.maximum(z, 0.0)
    tab_ref[0] = z @ Wsrc_ref[0]
    tab_ref[1] = z @ Wsrc_ref[1]
    dtab_ref[...] = z @ Wdst_ref[...]


def _z_from_acc(f0, f1, x0, x1, bc):
    def head(f, x, j):
        num = f[:, HD * j:HD * (j + 1)]
        den = x[:, j:j + 1] + 1e-16
        return num / den
    z = jnp.concatenate([head(f0, x0, 0), head(f0, x0, 1),
                         head(f1, x1, 0), head(f1, x1, 1)], axis=1) + bc
    return jnp.maximum(z, 0.0)


def _k2_body(f0_ref, f1_ref, x0_ref, x1_ref, bc_ref, Wsrc_ref, Wdst_ref,
             tab_ref, dtab_ref):
    z = _z_from_acc(f0_ref[0], f1_ref[0], x0_ref[0], x1_ref[0], bc_ref[...])
    tab_ref[0] = z @ Wsrc_ref[0]
    tab_ref[1] = z @ Wsrc_ref[1]
    dtab_ref[...] = z @ Wdst_ref[...]


def _k3_body(f0_ref, f1_ref, x0_ref, x1_ref, b_ref, s_ref, h_ref, g_ref,
             bc_ref, W1_ref, b1_ref, W2_ref, b2_ref, W3_ref, b3_ref,
             o_ref, pooled):
    i = pl.program_id(0)

    @pl.when(i == 0)
    def _init():
        pooled[...] = jnp.zeros_like(pooled)

    z = _z_from_acc(f0_ref[0], f1_ref[0], x0_ref[0], x1_ref[0], bc_ref[...])
    onehot = (b_ref[...] == lax.broadcasted_iota(jnp.int32, (1, B), 1)
              ).astype(jnp.float32)
    pooled[...] += lax.dot_general(onehot, z, (((0,), (0,)), ((), ())))

    @pl.when(i == NBLK - 1)
    def _final():
        comb = jnp.concatenate(
            [pooled[...], s_ref[...], h_ref[...], g_ref[...]], axis=1)
        o1 = jnp.maximum(comb @ W1_ref[...] + b1_ref[...], 0.0)
        o2 = jnp.maximum(o1 @ W2_ref[...] + b2_ref[...], 0.0)
        o_ref[...] = o2 @ W3_ref[...] + b3_ref[...]


def _edge_body(srcI_ref, dstI_ref, tab_ref, dtab_ref, outf_ref, outx_ref,
               sidx, sidx2, didx, didx8, srowsL, drowsL, orowsL, xrowsL,
               zbuf, zbufx, accf, accx, gsem, osem):
    c = lax.axis_index("c")
    sid = lax.axis_index("s")
    lane = lax.iota(jnp.int32, 16)
    par = jnp.bitwise_and(lane, 1)
    z16 = jnp.zeros((16,), jnp.float32)
    coff = c * NPAD

    def zb(r, cy):
        zbuf[r, pl.ds(0, 16)] = z16
        zbuf[r, pl.ds(16, 16)] = z16
        return cy
    lax.fori_loop(0, ZR, zb, 0)

    def zc(r, cy):
        pltpu.sync_copy(zbuf, accf.at[pl.ds(sid * RPT + r * ZR, ZR)])
        return cy
    lax.fori_loop(0, RPT // ZR, zc, 0)

    def zx(r, cy):
        zbufx[r, pl.ds(0, 16)] = z16
        return cy
    lax.fori_loop(0, ZRX, zx, 0)

    def zcx(r, cy):
        pltpu.sync_copy(zbufx, accx.at[pl.ds(sid * RPTX + r * ZRX, ZRX)])
        return cy
    lax.fori_loop(0, RPTX // ZRX, zcx, 0)
    plsc.subcore_barrier()

    def chunk(gb, carry):
        pltpu.sync_copy(srcI_ref.at[sid, gb], sidx)
        pltpu.sync_copy(dstI_ref.at[sid, gb], didx)

        def addrow(j, cy):
            def addk(k, cy2):
                sl = pl.ds(k * 16, 16)
                sidx2[j, sl] = sidx[j, sl] + coff
                didx8[j, sl] = lax.shift_right_logical(didx[j, sl], 3)
                return cy2
            return lax.fori_loop(0, G // 16, addk, cy)
        lax.fori_loop(0, 8, addrow, 0)

        def issue_gather(j):
            p = j % 2
            return (pltpu.async_copy(tab_ref.at[sidx2.at[j]], srowsL[p],
                                     gsem[p]),
                    pltpu.async_copy(dtab_ref.at[didx.at[j]], drowsL[p],
                                     gsem[p]))

        def compute_group(j):
            p = j % 2
            srows = srowsL[p]
            drows = drowsL[p]
            orows = orowsL[p]
            xrows = xrowsL[p]

            def block16(bq, cy2):
                dvec = didx[j, pl.ds(bq * 16, 16)]
                base = bq * 16
                for li in range(16):
                    e = base + li
                    kk2 = 2 * jnp.bitwise_and(dvec[li], 7)
                    v = srows[e, pl.ds(24, 16)]
                    w = drows[e, pl.ds(0, 16)]
                    a0 = v[8]
                    a1 = v[9]
                    d0 = jnp.where(c == 0, w[0], w[2])
                    d1 = jnp.where(c == 0, w[1], w[3])
                    sv = jnp.where(par == 0, a0 + d0, a1 + d1)
                    sv = jnp.maximum(sv, sv * 0.2)
                    exv = jnp.exp(sv)
                    e0 = exv[0]
                    e1 = exv[1]
                    orows[e, pl.ds(0, 16)] = srows[e, pl.ds(0, 16)] * e0
                    orows[e, pl.ds(16, 16)] = srows[e, pl.ds(16, 16)] * e1
                    xrows[e, pl.ds(0, 16)] = jnp.where(
                        lane == kk2, e0, jnp.where(lane == kk2 + 1, e1, 0.0))
                return cy2
            lax.fori_loop(0, G // 16, block16, 0)

        gnext = issue_gather(0)
        scat = [None, None]
        for j in range(8):
            p = j % 2
            gcur = gnext
            if j < 7:
                gnext = issue_gather(j + 1)
            gcur[0].wait()
            gcur[1].wait()
            if scat[p] is not None:
                scat[p][0].wait()
                scat[p][1].wait()
            compute_group(j)
            scat[p] = (pltpu.async_copy(orowsL[p], accf.at[didx.at[j]],
                                        osem[p], add=True),
                       pltpu.async_copy(xrowsL[p], accx.at[didx8.at[j]],
                                        osem[p], add=True))
        for p in range(2):
            scat[p][0].wait()
            scat[p][1].wait()
        return carry
    lax.fori_loop(0, NGB, chunk, 0)

    plsc.subcore_barrier()
    pltpu.sync_copy(accf.at[pl.ds(sid * RPT, RPT)],
                    outf_ref.at[c, pl.ds(sid * RPT, RPT)])
    pltpu.sync_copy(accx.at[pl.ds(sid * RPTX, RPTX)],
                    outx_ref.at[c, pl.ds(sid * RPTX, RPTX)])


def _edge_pass(srcI, dstI, tab_flat, dtab):
    mesh = plsc.VectorSubcoreMesh(core_axis_name="c", subcore_axis_name="s",
                                  num_cores=NC, num_subcores=NS)
    fn = pl.kernel(
        _edge_body,
        out_type=[
            jax.ShapeDtypeStruct((NC, NPAD, 32), jnp.float32),
            jax.ShapeDtypeStruct((NC, NPADX, 16), jnp.float32),
        ],
        mesh=mesh,
        compiler_params=pltpu.CompilerParams(use_tc_tiling_on_sc=False),
        scratch_types=[
            pltpu.VMEM((8, G), jnp.int32),
            pltpu.VMEM((8, G), jnp.int32),
            pltpu.VMEM((8, G), jnp.int32),
            pltpu.VMEM((8, G), jnp.int32),
            [pltpu.VMEM((G, RT), jnp.float32)] * 2,
            [pltpu.VMEM((G, RD), jnp.float32)] * 2,
            [pltpu.VMEM((G, 32), jnp.float32)] * 2,
            [pltpu.VMEM((G, 16), jnp.float32)] * 2,
            pltpu.VMEM((ZR, 32), jnp.float32),
            pltpu.VMEM((ZRX, 16), jnp.float32),
            pltpu.VMEM_SHARED((NPAD, 32), jnp.float32),
            pltpu.VMEM_SHARED((NPADX, 16), jnp.float32),
            [pltpu.SemaphoreType.DMA] * 2,
            [pltpu.SemaphoreType.DMA] * 2,
        ],
    )
    return fn(srcI, dstI, tab_flat, dtab)


def _node_tables_l0(x, batch2d, s, h, Wx, Wsh, b_in, Wsrc, Wdst):
    return pl.pallas_call(
        _k1_body,
        grid=(NBLK,),
        in_specs=[
            pl.BlockSpec((NB, D_IN), lambda i: (i, 0)),
            pl.BlockSpec((NB, 1), lambda i: (i, 0)),
            pl.BlockSpec((B, 32), lambda i: (0, 0)),
            pl.BlockSpec((B, 16), lambda i: (0, 0)),
            pl.BlockSpec((D_IN, HID), lambda i: (0, 0)),
            pl.BlockSpec((48, HID), lambda i: (0, 0)),
            pl.BlockSpec((1, HID), lambda i: (0, 0)),
            pl.BlockSpec((NC, HID, RT), lambda i: (0, 0, 0)),
            pl.BlockSpec((HID, RD), lambda i: (0, 0)),
        ],
        out_specs=[
            pl.BlockSpec((NC, NB, RT), lambda i: (0, i, 0)),
            pl.BlockSpec((NB, RD), lambda i: (i, 0)),
        ],
        out_shape=[
            jax.ShapeDtypeStruct((NC, NPAD, RT), jnp.float32),
            jax.ShapeDtypeStruct((NPAD, RD), jnp.float32),
        ],
    )(x, batch2d, s, h, Wx, Wsh, b_in, Wsrc, Wdst)


def _acc_specs():
    return [
        pl.BlockSpec((1, NB, 32), lambda i: (0, i, 0)),
        pl.BlockSpec((1, NB, 32), lambda i: (1, i, 0)),
        pl.BlockSpec((1, NB, 2), lambda i: (0, i, 0)),
        pl.BlockSpec((1, NB, 2), lambda i: (1, i, 0)),
    ]


def _node_tables_l1(accf, accx2, bc, Wsrc, Wdst):
    return pl.pallas_call(
        _k2_body,
        grid=(NBLK,),
        in_specs=_acc_specs() + [
            pl.BlockSpec((1, HID), lambda i: (0, 0)),
            pl.BlockSpec((NC, HID, RT), lambda i: (0, 0, 0)),
            pl.BlockSpec((HID, RD), lambda i: (0, 0)),
        ],
        out_specs=[
            pl.BlockSpec((NC, NB, RT), lambda i: (0, i, 0)),
            pl.BlockSpec((NB, RD), lambda i: (i, 0)),
        ],
        out_shape=[
            jax.ShapeDtypeStruct((NC, NPAD, RT), jnp.float32),
            jax.ShapeDtypeStruct((NPAD, RD), jnp.float32),
        ],
    )(accf, accf, accx2, accx2, bc, Wsrc, Wdst)


def _readout(accf, accx2, batch2d, s, h, g, bc, W1, b1, W2, b2, W3, b3):
    return pl.pallas_call(
        _k3_body,
        grid=(NBLK,),
        in_specs=_acc_specs() + [
            pl.BlockSpec((NB, 1), lambda i: (i, 0)),
            pl.BlockSpec((B, 32), lambda i: (0, 0)),
            pl.BlockSpec((B, 16), lambda i: (0, 0)),
            pl.BlockSpec((B, 8), lambda i: (0, 0)),
            pl.BlockSpec((1, HID), lambda i: (0, 0)),
            pl.BlockSpec((120, 256), lambda i: (0, 0)),
            pl.BlockSpec((1, 256), lambda i: (0, 0)),
            pl.BlockSpec((256, 128), lambda i: (0, 0)),
            pl.BlockSpec((1, 128), lambda i: (0, 0)),
            pl.BlockSpec((128, 1), lambda i: (0, 0)),
            pl.BlockSpec((1, 1), lambda i: (0, 0)),
        ],
        out_specs=pl.BlockSpec((B, 1), lambda i: (0, 0)),
        out_shape=jax.ShapeDtypeStruct((B, 1), jnp.float32),
        scratch_shapes=[pltpu.VMEM((B, HID), jnp.float32)],
    )(accf, accf, accx2, accx2, batch2d, s, h, g, bc,
      W1, b1, W2, b2, W3, b3)


def _layer_weights(Wc, a_s, a_d):
    Wc3 = Wc.reshape(HID, HEADS, HD)
    Asrc = jnp.einsum('khd,hd->kh', Wc3, a_s)
    Adst = jnp.einsum('khd,hd->kh', Wc3, a_d)
    z6 = jnp.zeros((HID, 6), jnp.float32)
    Wsrc = jnp.stack([
        jnp.concatenate([Wc[:, 0:32], Asrc[:, 0:2], z6], axis=1),
        jnp.concatenate([Wc[:, 32:64], Asrc[:, 2:4], z6], axis=1)], axis=0)
    Wdst = jnp.concatenate([Adst, jnp.zeros((HID, 12), jnp.float32)], axis=1)
    return Wsrc, Wdst


def kernel(x, edge_index, batch, s, h, g, W_in, b_in, Wc0, as0, ad0, bc0,
           Wc1, as1, ad1, bc1, W1, b1, W2, b2, W3, b3):
    pad = jnp.full((EPAD - E,), N, jnp.int32)
    srcI = jnp.concatenate([edge_index[0], pad]).reshape(NS, NGB, 8, G)
    dstI = jnp.concatenate([edge_index[1], pad]).reshape(NS, NGB, 8, G)
    batch2d = batch.reshape(N, 1)
    Wx = W_in[:D_IN]
    Wsh = W_in[D_IN:]
    Wsrc0, Wdst0 = _layer_weights(Wc0, as0, ad0)
    Wsrc1, Wdst1 = _layer_weights(Wc1, as1, ad1)

    tab0, dtab0 = _node_tables_l0(x, batch2d, s, h, Wx, Wsh,
                                  b_in.reshape(1, HID), Wsrc0, Wdst0)
    f0, x0 = _edge_pass(srcI, dstI, tab0.reshape(NC * NPAD, RT), dtab0)
    tab1, dtab1 = _node_tables_l1(f0, x0.reshape(NC, NPAD, 2),
                                  bc0.reshape(1, HID), Wsrc1, Wdst1)
    f1, x1 = _edge_pass(srcI, dstI, tab1.reshape(NC * NPAD, RT), dtab1)
    o = _readout(f1, x1.reshape(NC, NPAD, 2), batch2d, s, h, g,
                 bc1.reshape(1, HID), W1, b1.reshape(1, 256),
                 W2, b2.reshape(1, 128), W3, b3.reshape(1, 1))
    return o[:, 0]
